# speedup vs baseline: 9.0577x; 9.0577x over previous
import functools

import jax
import jax.numpy as jnp
from jax import lax
from jax.experimental import pallas as pl
from jax.experimental.pallas import tpu as pltpu
from jax.experimental.pallas import tpu_sc as plsc

_NUM_EMB = 100000
_DIM = 128
_BATCH = 4096
_SEQ = 200
_TOTAL = _BATCH * _SEQ

_NC = 2
_NS = 16
_NW = _NC * _NS
_PER_W = _TOTAL // _NW

_CHUNK = 128
_CPB = 2
_BUF_ROWS = _CHUNK * _CPB
_NIT = _PER_W // _BUF_ROWS
_CHUNKS_PER_W = _PER_W // _CHUNK


def _emb_body(x_hbm, w_hbm, out_hbm, idx_v, rows_v, g0, g1, o0, o1):
    wid = lax.axis_index("s") * _NC + lax.axis_index("c")
    cbase = wid * _CHUNKS_PER_W

    pltpu.sync_copy(x_hbm.at[pl.ds(cbase, _CHUNKS_PER_W)], idx_v)

    gsems = (g0, g1)
    osems = (o0, o1)

    def gather_desc(it, s, j):
        return pltpu.make_async_copy(
            w_hbm.at[idx_v.at[it * _CPB + j]],
            rows_v.at[s, pl.ds(j * _CHUNK, _CHUNK)],
            gsems[s])

    def fire_gather(it, s):
        for j in range(_CPB):
            gather_desc(it, s, j).start()

    def wait_gather(it, s):
        for j in range(_CPB):
            gather_desc(it, s, j).wait()

    def out_desc(it, s):
        roff = (cbase + it * _CPB) * _CHUNK
        return pltpu.make_async_copy(
            rows_v.at[s], out_hbm.at[pl.ds(roff, _BUF_ROWS)], osems[s])

    fire_gather(0, 0)
    fire_gather(1, 1)

    def step(k, carry):
        t = k * 2
        wait_gather(t, 0)
        out_desc(t, 0).start()
        wait_gather(t + 1, 1)
        out_desc(t + 1, 1).start()
        out_desc(t, 0).wait()
        fire_gather(t + 2, 0)
        out_desc(t + 1, 1).wait()
        fire_gather(t + 3, 1)
        return carry

    lax.fori_loop(0, (_NIT - 2) // 2, step, 0)

    t = _NIT - 2
    wait_gather(t, 0)
    out_desc(t, 0).start()
    wait_gather(t + 1, 1)
    out_desc(t + 1, 1).start()
    out_desc(t, 0).wait()
    out_desc(t + 1, 1).wait()


@functools.partial(
    pl.kernel,
    out_type=jax.ShapeDtypeStruct((_TOTAL, _DIM), jnp.float32),
    mesh=plsc.VectorSubcoreMesh(core_axis_name="c", subcore_axis_name="s"),
    scratch_types=[
        pltpu.VMEM((_CHUNKS_PER_W, _CHUNK), jnp.int32),
        pltpu.VMEM((2, _BUF_ROWS, _DIM), jnp.float32),
        pltpu.SemaphoreType.DMA,
        pltpu.SemaphoreType.DMA,
        pltpu.SemaphoreType.DMA,
        pltpu.SemaphoreType.DMA,
    ],
)
def _emb_lookup(x_hbm, w_hbm, out_hbm, idx_v, rows_v, g0, g1, o0, o1):
    _emb_body(x_hbm, w_hbm, out_hbm, idx_v, rows_v, g0, g1, o0, o1)


def kernel(x, weight):
    x2 = x.astype(jnp.int32).reshape(_TOTAL // _CHUNK, _CHUNK)
    out = _emb_lookup(x2, weight)
    return out.reshape(_BATCH, _SEQ, _DIM)


# speedup vs baseline: 9.1978x; 1.0155x over previous
import functools

import jax
import jax.numpy as jnp
from jax import lax
from jax.experimental import pallas as pl
from jax.experimental.pallas import tpu as pltpu
from jax.experimental.pallas import tpu_sc as plsc

_NUM_EMB = 100000
_DIM = 128
_BATCH = 4096
_SEQ = 200
_TOTAL = _BATCH * _SEQ

_NC = 2
_NS = 16
_NW = _NC * _NS
_PER_W = _TOTAL // _NW

_CHUNK = 128
_NBUF = 4
_NIT = _PER_W // _CHUNK


def _emb_body(x_hbm, w_hbm, out_hbm, idx_v, rows_v, gsems, osems):
    wid = lax.axis_index("s") * _NC + lax.axis_index("c")
    cbase = wid * _NIT

    pltpu.sync_copy(x_hbm.at[pl.ds(cbase, _NIT)], idx_v)

    def gather_desc(t, s):
        return pltpu.make_async_copy(
            w_hbm.at[idx_v.at[t]], rows_v.at[s], gsems[s])

    def out_desc(t, s):
        roff = (cbase + t) * _CHUNK
        return pltpu.make_async_copy(
            rows_v.at[s], out_hbm.at[pl.ds(roff, _CHUNK)], osems[s])

    gather_desc(0, 0).start()
    gather_desc(1, 1).start()
    gather_desc(2, 2).start()
    gather_desc(0, 0).wait()
    out_desc(0, 0).start()
    gather_desc(3, 3).start()
    gather_desc(1, 1).wait()
    out_desc(1, 1).start()

    def step(k, carry):
        for i in range(_NBUF):
            t = k * _NBUF + 2 + i
            s = (2 + i) % _NBUF
            sf = i % _NBUF
            out_desc(t - 2, sf).wait()
            gather_desc(t + 2, sf).start()
            gather_desc(t, s).wait()
            out_desc(t, s).start()
        return carry

    lax.fori_loop(0, (_NIT - 4) // _NBUF, step, 0)

    t = _NIT - 2
    out_desc(t - 2, 0).wait()
    gather_desc(t, 2).wait()
    out_desc(t, 2).start()
    out_desc(t - 1, 1).wait()
    gather_desc(t + 1, 3).wait()
    out_desc(t + 1, 3).start()
    out_desc(t, 2).wait()
    out_desc(t + 1, 3).wait()


@functools.partial(
    pl.kernel,
    out_type=jax.ShapeDtypeStruct((_TOTAL, _DIM), jnp.float32),
    mesh=plsc.VectorSubcoreMesh(core_axis_name="c", subcore_axis_name="s"),
    scratch_types=[
        pltpu.VMEM((_NIT, _CHUNK), jnp.int32),
        pltpu.VMEM((_NBUF, _CHUNK, _DIM), jnp.float32),
        pltpu.SemaphoreType.DMA,
        pltpu.SemaphoreType.DMA,
        pltpu.SemaphoreType.DMA,
        pltpu.SemaphoreType.DMA,
        pltpu.SemaphoreType.DMA,
        pltpu.SemaphoreType.DMA,
        pltpu.SemaphoreType.DMA,
        pltpu.SemaphoreType.DMA,
    ],
)
def _emb_lookup(x_hbm, w_hbm, out_hbm, idx_v, rows_v,
                g0, g1, g2, g3, o0, o1, o2, o3):
    _emb_body(x_hbm, w_hbm, out_hbm, idx_v, rows_v,
              (g0, g1, g2, g3), (o0, o1, o2, o3))


def kernel(x, weight):
    x2 = x.astype(jnp.int32).reshape(_TOTAL // _CHUNK, _CHUNK)
    out = _emb_lookup(x2, weight)
    return out.reshape(_BATCH, _SEQ, _DIM)
